# baseline (speedup 1.0000x reference)
import jax
import jax.numpy as jnp
from jax import lax
from jax.experimental import pallas as pl
from jax.experimental.pallas import tpu as pltpu
from jax.experimental.pallas import tpu_sc as plsc

_N = 100
_NP = 128
_E = 6400
_EROWS = 64
_EC = 104
_ACC = _NP * _NP
_SLICE = _ACC // 16
_F32 = jnp.float32
_PH = lax.Precision.HIGHEST


def _sc_body(fidx_hbm, w_hbm, out_hbm, idx_v, w_v, z_v, acc_sh):
    c = lax.axis_index("c")
    s = lax.axis_index("s")
    for i in range(_SLICE // 16):
        z_v[pl.ds(i * 16, 16)] = jnp.zeros((16,), _F32)
    pltpu.sync_copy(z_v, acc_sh.at[pl.ds(s * _SLICE, _SLICE)])
    plsc.subcore_barrier()
    r0 = (c * 16 + s) * 2
    pltpu.sync_copy(fidx_hbm.at[pl.ds(r0, 2)], idx_v)
    pltpu.sync_copy(w_hbm.at[pl.ds(r0, 2)], w_v)
    for j in range(2):
        pltpu.sync_copy(w_v.at[j], acc_sh.at[idx_v.at[j]], add=True)
    plsc.subcore_barrier()
    pltpu.sync_copy(acc_sh.at[pl.ds(s * _SLICE, _SLICE)],
                    out_hbm.at[c, pl.ds(s * _SLICE, _SLICE)])


_sc_build = pl.kernel(
    _sc_body,
    mesh=plsc.VectorSubcoreMesh(core_axis_name="c", subcore_axis_name="s"),
    out_type=jax.ShapeDtypeStruct((2, _ACC), _F32),
    scratch_types=[
        pltpu.VMEM((2, _EC), jnp.int32),
        pltpu.VMEM((2, _EC), _F32),
        pltpu.VMEM((_SLICE,), _F32),
        pltpu.VMEM_SHARED((_ACC,), _F32),
    ],
)


def _dense_body(a2_ref, xv_ref, wg1_ref, bg1_ref,
                wg2_ref, bg2_ref, wg3_ref, bg3_ref, gam_ref, bet_ref,
                w1_ref, b1_ref, w2g_ref, b2_ref, out_ref):
    at = a2_ref[0] + a2_ref[1]
    deg = jnp.sum(at, axis=1, keepdims=True) + 1.0
    dinv = lax.rsqrt(deg)

    def gcn(h, w, b):
        z = jnp.dot(h, w, preferred_element_type=_F32)
        zh = dinv * z
        agg = jnp.dot(at, zh, precision=_PH, preferred_element_type=_F32) + zh
        return jax.nn.relu(dinv * agg + b)

    h = gcn(xv_ref[...], wg1_ref[...], bg1_ref[...])
    h = gcn(h, wg2_ref[...], bg2_ref[...])
    h = gcn(h, wg3_ref[...], bg3_ref[...])

    rmask = (lax.broadcasted_iota(jnp.int32, (_NP, 1), 0) < _N).astype(_F32)
    mean = jnp.sum(h * rmask, axis=0, keepdims=True) * (1.0 / _N)
    diff = h - mean
    var = jnp.sum(diff * diff * rmask, axis=0, keepdims=True) * (1.0 / _N)
    hn = diff * lax.rsqrt(var + 1e-5) * gam_ref[...] + bet_ref[...]

    l = jax.nn.relu(jnp.dot(hn, w1_ref[...],
                            preferred_element_type=_F32) + b1_ref[...])
    g = lax.dot_general(l, w2g_ref[...], (((0,), (0,)), ((), ())),
                        preferred_element_type=_F32)
    acc = b2_ref[...]
    for c in range(10):
        acc = acc + g[c:c + 1, c * 128:(c + 1) * 128]
    out_ref[...] = acc


def kernel(x, edge_index, edge_attr, Wg1, bg1, Wg2, bg2, Wg3, bg3,
           gamma, beta, W1, b1, W2, b2):
    src = edge_index[0, 0].astype(jnp.int32)
    dst = edge_index[0, 1].astype(jnp.int32)
    fidx = (dst * _NP + src).reshape(_EROWS, _E // _EROWS)
    fidx_p = jnp.pad(fidx, ((0, 0), (0, _EC - _E // _EROWS)))
    w_p = jnp.pad(edge_attr[0].reshape(_EROWS, _E // _EROWS),
                  ((0, 0), (0, _EC - _E // _EROWS)))
    a2 = _sc_build(fidx_p, w_p).reshape(2, _NP, _NP)

    xv = jnp.pad(x[0].reshape(_N, 128), ((0, _NP - _N), (0, 0)))
    w2g = jnp.pad(W2.reshape(_N, 1280), ((0, _NP - _N), (0, 0)))
    out = pl.pallas_call(
        _dense_body,
        out_shape=jax.ShapeDtypeStruct((1, 128), _F32),
    )(a2, xv,
      Wg1, bg1.reshape(1, 64), Wg2, bg2.reshape(1, 128),
      Wg3, bg3.reshape(1, 256), gamma.reshape(1, 256), beta.reshape(1, 256),
      W1, b1.reshape(1, 10), w2g, b2.reshape(1, 128))
    return out.reshape(128)
